# baseline (speedup 1.0000x reference)
import jax
import jax.numpy as jnp
from jax.experimental import pallas as pl

_BR = 4160


def _body(x_ref, m_ref, tw_ref, tb_ref, vw_ref, vb_ref, emp_ref, unm_ref, o_ref):
    v = x_ref[:, 0:1]
    t = x_ref[:, 1:2]
    m = m_ref[:]
    inv = jnp.isnan(v)
    valid = jnp.logical_and(m, jnp.logical_not(inv))
    v0 = jnp.where(valid, v, 0.0)
    tw = tw_ref[:]
    tb = tb_ref[:]
    base_valid = tb + vb_ref[:]
    base_unm = tb + unm_ref[:]
    base_emp = tb + emp_ref[:]
    base = jnp.where(m, jnp.where(inv, base_emp, base_valid), base_unm)
    o_ref[:] = t * tw + v0 * vw_ref[:] + base


def kernel(x, monitor_mask, time_emb_w, time_emb_b, value_emb_w, value_emb_b,
           empty_token, unmonitored_token):
    N, T, P, _ = x.shape
    D = time_emb_w.shape[-1]
    R = N * T * P
    xf = x.reshape(R, 2)
    mf = monitor_mask.reshape(R, 1)
    grid = R // _BR
    out = pl.pallas_call(
        _body,
        grid=(grid,),
        in_specs=[
            pl.BlockSpec((_BR, 2), lambda i: (i, 0)),
            pl.BlockSpec((_BR, 1), lambda i: (i, 0)),
            pl.BlockSpec((1, D), lambda i: (0, 0)),
            pl.BlockSpec((1, D), lambda i: (0, 0)),
            pl.BlockSpec((1, D), lambda i: (0, 0)),
            pl.BlockSpec((1, D), lambda i: (0, 0)),
            pl.BlockSpec((1, D), lambda i: (0, 0)),
            pl.BlockSpec((1, D), lambda i: (0, 0)),
        ],
        out_specs=pl.BlockSpec((_BR, D), lambda i: (i, 0)),
        out_shape=jax.ShapeDtypeStruct((R, D), jnp.float32),
    )(xf, mf, time_emb_w, time_emb_b, value_emb_w, value_emb_b,
      empty_token.reshape(1, D), unmonitored_token.reshape(1, D))
    return out.reshape(N, T, P, D)
